# baseline (speedup 1.0000x reference)
import functools

import jax
import jax.numpy as jnp
from jax import lax
from jax.experimental import pallas as pl
from jax.experimental.pallas import tpu as pltpu
from jax.experimental.pallas import tpu_sc as plsc

_NC = 2
_NS = 16
_NW = _NC * _NS
_L = 16

_CHUNK = 16
_UNROLL = 8


@functools.lru_cache(maxsize=None)
def _build(n_rows: int, d: int):
    r_per_w = n_rows // _NW
    n_chunks = r_per_w // _CHUNK
    n_pairs = n_chunks // 2
    n_cb = d // _L
    mesh = plsc.VectorSubcoreMesh(core_axis_name="c", subcore_axis_name="s")

    @functools.partial(
        pl.kernel,
        mesh=mesh,
        compiler_params=pltpu.CompilerParams(needs_layout_passes=False),
        out_type=jax.ShapeDtypeStruct((n_rows * d,), jnp.float32),
        scratch_types=[
            pltpu.VMEM((r_per_w,), jnp.int32),
            pltpu.VMEM((2, d), jnp.float32),
            pltpu.VMEM((_CHUNK * d,), jnp.float32),
            pltpu.VMEM((_CHUNK * d,), jnp.float32),
            pltpu.SemaphoreType.DMA,
            pltpu.SemaphoreType.DMA,
        ],
    )
    def emb(idx_hbm, table_hbm, out_hbm, idx_v, tab_v, buf0, buf1, sem0, sem1):
        sid = lax.axis_index("s")
        wid = sid * _NC + lax.axis_index("c")
        base = wid * r_per_w

        pltpu.sync_copy(table_hbm, tab_v)
        pltpu.sync_copy(idx_hbm.at[pl.ds(base, r_per_w)], idx_v)
        plsc.subcore_barrier()

        ci = lax.iota(jnp.int32, _L)
        rvoff = ci * d

        def fill(buf, c):
            rv = idx_v[pl.ds(c * _CHUNK, _L)]

            def col_body(j, carry):
                for u in range(_UNROLL):
                    col = j * _UNROLL + u
                    cs = jnp.full((_L,), col, jnp.int32)
                    val = plsc.load_gather(tab_v, [rv, cs])
                    plsc.store_scatter(buf, [rvoff + col], val)
                return carry

            lax.fori_loop(0, d // _UNROLL, col_body, 0)

        def out_slice(c):
            return out_hbm.at[pl.ds((base + c * _CHUNK) * d, _CHUNK * d)]

        fill(buf0, 0)
        pltpu.async_copy(buf0, out_slice(0), sem0)
        fill(buf1, 1)
        pltpu.async_copy(buf1, out_slice(1), sem1)

        def pair_body(p, carry):
            c0 = 2 * p
            pltpu.make_async_copy(buf0, out_slice(0), sem0).wait()
            fill(buf0, c0)
            pltpu.async_copy(buf0, out_slice(c0), sem0)
            pltpu.make_async_copy(buf1, out_slice(1), sem1).wait()
            fill(buf1, c0 + 1)
            pltpu.async_copy(buf1, out_slice(c0 + 1), sem1)
            return carry

        lax.fori_loop(1, n_pairs, pair_body, 0)

        pltpu.make_async_copy(buf0, out_slice(0), sem0).wait()
        pltpu.make_async_copy(buf1, out_slice(1), sem1).wait()

    return emb


def kernel(styles, table):
    b, s = styles.shape
    d = table.shape[1]
    idx = styles.reshape(-1).astype(jnp.int32)
    out = _build(b * s, d)(idx, table)
    return out.reshape(b, s, d)


# speedup vs baseline: 7.2277x; 7.2277x over previous
import functools

import jax
import jax.numpy as jnp
from jax import lax
from jax.experimental import pallas as pl
from jax.experimental.pallas import tpu as pltpu
from jax.experimental.pallas import tpu_sc as plsc

_NC = 2
_NS = 16
_NW = _NC * _NS
_L = 16

_CHUNK = 16


@functools.lru_cache(maxsize=None)
def _build(n_rows: int, d: int):
    r_per_w = n_rows // _NW
    n_chunks = r_per_w // _CHUNK
    mesh = plsc.VectorSubcoreMesh(core_axis_name="c", subcore_axis_name="s")

    @functools.partial(
        pl.kernel,
        mesh=mesh,
        compiler_params=pltpu.CompilerParams(needs_layout_passes=False),
        out_type=jax.ShapeDtypeStruct((n_rows * d,), jnp.float32),
        scratch_types=[
            pltpu.VMEM((r_per_w,), jnp.int32),
            pltpu.VMEM((2 * d,), jnp.float32),
            pltpu.SemaphoreType.DMA,
        ],
    )
    def emb(idx_hbm, table_hbm, out_hbm, idx_v, tab_v, sem):
        sid = lax.axis_index("s")
        wid = sid * _NC + lax.axis_index("c")
        base = wid * r_per_w

        pltpu.sync_copy(table_hbm, tab_v)
        pltpu.sync_copy(idx_hbm.at[pl.ds(base, r_per_w)], idx_v)

        ci = lax.iota(jnp.int32, _L)

        def fire(c):
            rv = idx_v[pl.ds(c * _CHUNK, _L)]
            for r in range(_CHUNK):
                iv = jnp.max(jnp.where(ci == r, rv, 0))
                pltpu.async_copy(
                    tab_v.at[pl.ds(iv * d, d)],
                    out_hbm.at[pl.ds((base + c * _CHUNK + r) * d, d)],
                    sem,
                )

        def drain():
            for _ in range(_CHUNK):
                pltpu.make_async_copy(
                    tab_v.at[pl.ds(0, d)], out_hbm.at[pl.ds(0, d)], sem
                ).wait()

        fire(0)

        def body(c, carry):
            fire(c)
            drain()
            return carry

        lax.fori_loop(1, n_chunks, body, 0)
        drain()

    return emb


def kernel(styles, table):
    b, s = styles.shape
    d = table.shape[1]
    idx = styles.reshape(-1).astype(jnp.int32)
    out = _build(b * s, d)(idx, table.reshape(-1))
    return out.reshape(b, s, d)
